# baseline (speedup 1.0000x reference)
import jax
import jax.numpy as jnp
from jax.experimental import pallas as pl

_MAXLEN = 8192
_EMBED = 128
_BATCH = 4
_BBLK = 2
_TBLK = 4096


def _add_body(x_ref, p_ref, o_ref):
    o_ref[...] = x_ref[...] + p_ref[...][None, :, :]


def kernel(x, pos_table):
    return pl.pallas_call(
        _add_body,
        grid=(_MAXLEN // _TBLK, _BATCH // _BBLK),
        in_specs=[
            pl.BlockSpec((_BBLK, _TBLK, _EMBED), lambda t, b: (b, t, 0)),
            pl.BlockSpec((_TBLK, _EMBED), lambda t, b: (t, 0)),
        ],
        out_specs=pl.BlockSpec((_BBLK, _TBLK, _EMBED), lambda t, b: (b, t, 0)),
        out_shape=jax.ShapeDtypeStruct((_BATCH, _MAXLEN, _EMBED), jnp.float32),
    )(x, pos_table)


# speedup vs baseline: 1.0878x; 1.0878x over previous
import jax
import jax.numpy as jnp
from jax.experimental import pallas as pl

_MAXLEN = 8192
_EMBED = 128
_BATCH = 4
_BBLK = 2


def _add_body(x_ref, p_ref, o_ref):
    o_ref[...] = x_ref[...] + p_ref[...][None, :, :]


def kernel(x, pos_table):
    return pl.pallas_call(
        _add_body,
        grid=(_BATCH // _BBLK,),
        in_specs=[
            pl.BlockSpec((_BBLK, _MAXLEN, _EMBED), lambda b: (b, 0, 0)),
            pl.BlockSpec((_MAXLEN, _EMBED), lambda b: (0, 0)),
        ],
        out_specs=pl.BlockSpec((_BBLK, _MAXLEN, _EMBED), lambda b: (b, 0, 0)),
        out_shape=jax.ShapeDtypeStruct((_BATCH, _MAXLEN, _EMBED), jnp.float32),
    )(x, pos_table)
